# baseline (speedup 1.0000x reference)
import functools

import jax
import jax.numpy as jnp
from jax.experimental import pallas as pl

_N = 10000
_K = 32
_H = 8


def _fc0_kernel(x_ref, w_ref, b_ref, o_ref):
    o_ref[...] = x_ref[...] @ w_ref[...] + b_ref[...]


def _fc0(feat, W, b):
    return pl.pallas_call(
        _fc0_kernel,
        out_shape=jax.ShapeDtypeStruct((feat.shape[0], W.shape[0]), jnp.float32),
    )(feat, W.T, b[None, :])


def _bn1d(x, p, eps=1e-5):
    m = jnp.mean(x, axis=0, keepdims=True)
    v = jnp.var(x, axis=0, keepdims=True)
    return p["g"] * (x - m) / jnp.sqrt(v + eps) + p["be"]


def _bn2d(x, p, eps=1e-5):
    m = jnp.mean(x, axis=(0, 2, 3), keepdims=True)
    v = jnp.var(x, axis=(0, 2, 3), keepdims=True)
    return p["g"][None, :, None, None] * (x - m) / jnp.sqrt(v + eps) + p["be"][None, :, None, None]


def _conv1x1(x, p):
    return jnp.einsum('bchw,oc->bohw', x, p["W"]) + p["b"][None, :, None, None]


def _lrelu(x):
    return jax.nn.leaky_relu(x, 0.01)


def _batch_knn(xyz, k, chunk=1000):
    n = xyz.shape[1]
    sq = jnp.sum(xyz * xyz, axis=-1)
    idxs = []
    for s in range(0, n, chunk):
        q = xyz[:, s:s + chunk]
        d = jnp.sum(q * q, -1)[:, :, None] - 2.0 * jnp.einsum('bcd,bnd->bcn', q, xyz) + sq[:, None, :]
        _, idx = jax.lax.top_k(-d, k)
        idxs.append(idx)
    return jnp.concatenate(idxs, axis=1)


def kernel(x, p):
    Bb, dims_in, n = x.shape
    K = _K
    H = _H
    og_xyz = x[:, 9:12, :]
    input_xyz = jnp.reshape(og_xyz, (Bb, n, 3))
    feat = jnp.reshape(x, (-1, dims_in))
    feat = _lrelu(_bn1d(_fc0(feat, p["fc0"]["W"], p["fc0"]["b"]), p["bn0"]))
    feat = jnp.transpose(jnp.reshape(feat, (Bb, n, 1, 32)), (0, 3, 2, 1))
    neigh_idx = jnp.reshape(_batch_knn(input_xyz, K), (Bb, n * K))

    f = jax.nn.relu(_bn2d(_conv1x1(feat, p["c1"]), p["b1"]))
    f = jnp.transpose(f, (0, 3, 2, 1))
    neigh_xyz = jnp.take_along_axis(input_xyz, neigh_idx[:, :, None], axis=1)
    neigh_feat = jnp.take_along_axis(f, neigh_idx[:, :, None, None], axis=1)
    tile_feat = jnp.transpose(jnp.tile(f, (1, 1, K, 1)), (0, 3, 2, 1))
    tile_xyz = jnp.transpose(jnp.tile(input_xyz[:, :, None, :], (1, 1, K, 1)), (0, 3, 2, 1))
    neigh_xyz = jnp.reshape(neigh_xyz, (Bb, 3, K, n))
    neigh_feat = jnp.reshape(neigh_feat, (Bb, H, K, n))

    feat_info = jnp.concatenate([neigh_feat - tile_feat, tile_feat], axis=1)
    nxo = jax.nn.relu(_bn2d(_conv1x1(feat_info, p["c2"]), p["b2"]))
    shifted_neigh_xyz = neigh_xyz + nxo
    xyz_info = jnp.concatenate([neigh_xyz - tile_xyz, shifted_neigh_xyz, tile_xyz], axis=1)
    nfo = jax.nn.relu(_bn2d(_conv1x1(xyz_info, p["c3"]), p["b3"]))
    shifted_neigh_feat = neigh_feat + nfo
    xyz_enc = jax.nn.relu(_bn2d(_conv1x1(xyz_info, p["c4"]), p["b4"]))
    feat_info2 = jnp.concatenate([shifted_neigh_feat, feat_info], axis=1)
    feat_enc = jax.nn.relu(_bn2d(_conv1x1(feat_info2, p["c5"]), p["b5"]))
    overall = jnp.concatenate([xyz_enc, feat_enc], axis=1)
    kw = jax.nn.softmax(_conv1x1(overall, p["c6"]), axis=2)
    wsum = jnp.sum(overall * kw, axis=2, keepdims=True)
    omax = jnp.max(overall, axis=2, keepdims=True)
    enc = jnp.concatenate([omax, wsum], axis=1)
    enc = jax.nn.relu(_bn2d(_conv1x1(enc, p["c7"]), p["b7"]))
    out_feat = _lrelu(_bn2d(_conv1x1(enc, p["c8"]), p["b8"]))

    f1 = _lrelu(_bn2d(_conv1x1(out_feat, p["m1"]), p["mb1"]))
    f2 = jnp.concatenate([f1, out_feat], axis=1)
    f2 = jax.nn.relu(_bn2d(_conv1x1(f2, p["up1"]), p["ub1"]))
    f3 = jax.nn.relu(_bn2d(_conv1x1(f2, p["m3"]), p["mb3"]))
    f4 = jax.nn.relu(_bn2d(_conv1x1(f3, p["m4"]), p["mb4"]))
    out = _conv1x1(f4, p["m5"])
    return out


# speedup vs baseline: 14.5501x; 14.5501x over previous
import functools

import jax
import jax.numpy as jnp
import numpy as np
from jax.experimental import pallas as pl
from jax.experimental.pallas import tpu as pltpu

_N = 10000
_NPAD = 10240
_K = 32
_H = 8
_QB = 256


def _knn_body(q_ref, xt_ref, xsq_ref, out_ref, d_ref):
    qsq = q_ref[:, 8:9]
    g = jnp.dot(q_ref[...].astype(jnp.bfloat16), xt_ref[...].astype(jnp.bfloat16),
                preferred_element_type=jnp.float32)
    d_ref[...] = qsq - 2.0 * g + xsq_ref[...]
    iota = jax.lax.broadcasted_iota(jnp.int32, (_QB, _NPAD), 1)
    d = d_ref[...]
    for i in range(_K):
        m = jnp.min(d, axis=1, keepdims=True)
        eq = d == m
        idx = jnp.min(jnp.where(eq, iota, _NPAD), axis=1)
        out_ref[i, :] = idx
        d = jnp.where(iota == idx[:, None], jnp.float32(np.inf), d)


def _knn_pallas(xyz):
    sq = jnp.sum(xyz * xyz, axis=-1)
    q = jnp.zeros((_NPAD, 16), jnp.float32)
    q = q.at[:_N, :3].set(xyz)
    q = q.at[_N:, :3].set(1e9)
    q = q.at[:_N, 8].set(sq)
    q = q.at[_N:, 8].set(3e18)
    xt = jnp.zeros((16, _NPAD), jnp.float32)
    xt = xt.at[:3, :_N].set(xyz.T)
    xt = xt.at[:3, _N:].set(1e9)
    xsq = q[:, 8][None, :]
    out = pl.pallas_call(
        _knn_body,
        grid=(_NPAD // _QB,),
        in_specs=[
            pl.BlockSpec((_QB, 16), lambda i: (i, 0)),
            pl.BlockSpec((16, _NPAD), lambda i: (0, 0)),
            pl.BlockSpec((1, _NPAD), lambda i: (0, 0)),
        ],
        out_specs=pl.BlockSpec((_K, _QB), lambda i: (0, i)),
        out_shape=jax.ShapeDtypeStruct((_K, _NPAD), jnp.int32),
        scratch_shapes=[pltpu.VMEM((_QB, _NPAD), jnp.float32)],
    )(q, xt, xsq)
    return out[:, :_N].T


def _fc0_kernel(x_ref, w_ref, b_ref, o_ref):
    o_ref[...] = x_ref[...] @ w_ref[...] + b_ref[...]


def _fc0(feat, W, b):
    return pl.pallas_call(
        _fc0_kernel,
        out_shape=jax.ShapeDtypeStruct((feat.shape[0], W.shape[0]), jnp.float32),
    )(feat, W.T, b[None, :])


def _bn1d(x, p, eps=1e-5):
    m = jnp.mean(x, axis=0, keepdims=True)
    v = jnp.var(x, axis=0, keepdims=True)
    return p["g"] * (x - m) / jnp.sqrt(v + eps) + p["be"]


def _bn2d(x, p, eps=1e-5):
    m = jnp.mean(x, axis=(0, 2, 3), keepdims=True)
    v = jnp.var(x, axis=(0, 2, 3), keepdims=True)
    return p["g"][None, :, None, None] * (x - m) / jnp.sqrt(v + eps) + p["be"][None, :, None, None]


def _conv1x1(x, p):
    return jnp.einsum('bchw,oc->bohw', x, p["W"]) + p["b"][None, :, None, None]


def _lrelu(x):
    return jax.nn.leaky_relu(x, 0.01)


def _batch_knn(xyz, k, chunk=1000):
    n = xyz.shape[1]
    sq = jnp.sum(xyz * xyz, axis=-1)
    idxs = []
    for s in range(0, n, chunk):
        q = xyz[:, s:s + chunk]
        d = jnp.sum(q * q, -1)[:, :, None] - 2.0 * jnp.einsum('bcd,bnd->bcn', q, xyz) + sq[:, None, :]
        _, idx = jax.lax.top_k(-d, k)
        idxs.append(idx)
    return jnp.concatenate(idxs, axis=1)


def kernel(x, p):
    Bb, dims_in, n = x.shape
    K = _K
    H = _H
    og_xyz = x[:, 9:12, :]
    input_xyz = jnp.reshape(og_xyz, (Bb, n, 3))
    feat = jnp.reshape(x, (-1, dims_in))
    feat = _lrelu(_bn1d(_fc0(feat, p["fc0"]["W"], p["fc0"]["b"]), p["bn0"]))
    feat = jnp.transpose(jnp.reshape(feat, (Bb, n, 1, 32)), (0, 3, 2, 1))
    neigh_idx = jnp.reshape(_knn_pallas(input_xyz[0])[None], (Bb, n * K))

    f = jax.nn.relu(_bn2d(_conv1x1(feat, p["c1"]), p["b1"]))
    f = jnp.transpose(f, (0, 3, 2, 1))
    neigh_xyz = jnp.take_along_axis(input_xyz, neigh_idx[:, :, None], axis=1)
    neigh_feat = jnp.take_along_axis(f, neigh_idx[:, :, None, None], axis=1)
    tile_feat = jnp.transpose(jnp.tile(f, (1, 1, K, 1)), (0, 3, 2, 1))
    tile_xyz = jnp.transpose(jnp.tile(input_xyz[:, :, None, :], (1, 1, K, 1)), (0, 3, 2, 1))
    neigh_xyz = jnp.reshape(neigh_xyz, (Bb, 3, K, n))
    neigh_feat = jnp.reshape(neigh_feat, (Bb, H, K, n))

    feat_info = jnp.concatenate([neigh_feat - tile_feat, tile_feat], axis=1)
    nxo = jax.nn.relu(_bn2d(_conv1x1(feat_info, p["c2"]), p["b2"]))
    shifted_neigh_xyz = neigh_xyz + nxo
    xyz_info = jnp.concatenate([neigh_xyz - tile_xyz, shifted_neigh_xyz, tile_xyz], axis=1)
    nfo = jax.nn.relu(_bn2d(_conv1x1(xyz_info, p["c3"]), p["b3"]))
    shifted_neigh_feat = neigh_feat + nfo
    xyz_enc = jax.nn.relu(_bn2d(_conv1x1(xyz_info, p["c4"]), p["b4"]))
    feat_info2 = jnp.concatenate([shifted_neigh_feat, feat_info], axis=1)
    feat_enc = jax.nn.relu(_bn2d(_conv1x1(feat_info2, p["c5"]), p["b5"]))
    overall = jnp.concatenate([xyz_enc, feat_enc], axis=1)
    kw = jax.nn.softmax(_conv1x1(overall, p["c6"]), axis=2)
    wsum = jnp.sum(overall * kw, axis=2, keepdims=True)
    omax = jnp.max(overall, axis=2, keepdims=True)
    enc = jnp.concatenate([omax, wsum], axis=1)
    enc = jax.nn.relu(_bn2d(_conv1x1(enc, p["c7"]), p["b7"]))
    out_feat = _lrelu(_bn2d(_conv1x1(enc, p["c8"]), p["b8"]))

    f1 = _lrelu(_bn2d(_conv1x1(out_feat, p["m1"]), p["mb1"]))
    f2 = jnp.concatenate([f1, out_feat], axis=1)
    f2 = jax.nn.relu(_bn2d(_conv1x1(f2, p["up1"]), p["ub1"]))
    f3 = jax.nn.relu(_bn2d(_conv1x1(f2, p["m3"]), p["mb3"]))
    f4 = jax.nn.relu(_bn2d(_conv1x1(f3, p["m4"]), p["mb4"]))
    out = _conv1x1(f4, p["m5"])
    return out
